# baseline (speedup 1.0000x reference)
import jax
import jax.numpy as jnp
from jax.experimental import pallas as pl
from jax.experimental.pallas import tpu as pltpu

B, H, L, D = 4, 16, 2048, 128
_CHROWS = 4
_NCH = (B * H) // _CHROWS
_NB = 3


def _fill_kernel(
    k_hbm, v_hbm, k_out, v_out, mask_ref,
    kbuf, vbuf, semki, semvi, semko, semvo,
):
    mask_ref[...] = jnp.ones_like(mask_ref)

    def in_copy(c, hbm, buf, sem):
        b = c % _NB
        return pltpu.make_async_copy(
            hbm.at[pl.ds(c * _CHROWS, _CHROWS)], buf.at[b], sem.at[b]
        )

    def out_copy(c, buf, hbm, sem):
        b = c % _NB
        return pltpu.make_async_copy(
            buf.at[b], hbm.at[pl.ds(c * _CHROWS, _CHROWS)], sem.at[b]
        )

    ko, vo = {}, {}
    ki, vi = {}, {}
    for c in range(_NB):
        ki[c] = in_copy(c, k_hbm, kbuf, semki)
        vi[c] = in_copy(c, v_hbm, vbuf, semvi)
        ki[c].start()
        vi[c].start()
    waited = set()
    for c in range(_NCH):
        ki[c].wait()
        ko[c] = out_copy(c, kbuf, k_out, semko)
        ko[c].start()
        vi[c].wait()
        vo[c] = out_copy(c, vbuf, v_out, semvo)
        vo[c].start()
        d = c - 2
        if d >= 0 and d + _NB < _NCH:
            ko[d].wait()
            vo[d].wait()
            waited.add(d)
            ki[d + _NB] = in_copy(d + _NB, k_hbm, kbuf, semki)
            vi[d + _NB] = in_copy(d + _NB, v_hbm, vbuf, semvi)
            ki[d + _NB].start()
            vi[d + _NB].start()
    for c in range(_NCH):
        if c not in waited:
            ko[c].wait()
            vo[c].wait()


def kernel(k_val, v_val, input_pos, is_prefill, k_cache, v_cache, pos, mask):
    del input_pos, is_prefill, k_cache, v_cache, pos
    kv3 = (B * H, L, D)
    k3 = k_val.reshape(kv3)
    v3 = v_val.reshape(kv3)
    mask3 = (B * H, 1, L)
    k_out, v_out, mask_out = pl.pallas_call(
        _fill_kernel,
        in_specs=[
            pl.BlockSpec(memory_space=pl.ANY),
            pl.BlockSpec(memory_space=pl.ANY),
        ],
        out_specs=[
            pl.BlockSpec(memory_space=pl.ANY),
            pl.BlockSpec(memory_space=pl.ANY),
            pl.BlockSpec(memory_space=pltpu.VMEM),
        ],
        out_shape=[
            jax.ShapeDtypeStruct(kv3, k_val.dtype),
            jax.ShapeDtypeStruct(kv3, v_val.dtype),
            jax.ShapeDtypeStruct(mask3, jnp.bool_),
        ],
        scratch_shapes=[
            pltpu.VMEM((_NB, _CHROWS, L, D), jnp.float32),
            pltpu.VMEM((_NB, _CHROWS, L, D), jnp.float32),
            pltpu.SemaphoreType.DMA((_NB,)),
            pltpu.SemaphoreType.DMA((_NB,)),
            pltpu.SemaphoreType.DMA((_NB,)),
            pltpu.SemaphoreType.DMA((_NB,)),
        ],
    )(k3, v3)
    return (
        k_out.reshape(B, H, L, D),
        v_out.reshape(B, H, L, D),
        mask_out.reshape(B, H, 1, L),
    )


# speedup vs baseline: 1.0748x; 1.0748x over previous
import jax
import jax.numpy as jnp
from jax.experimental import pallas as pl
from jax.experimental.pallas import tpu as pltpu

B, H, L, D = 4, 16, 2048, 128
_RB = 8
_G = (B * H) // _RB


def _fill_kernel(k_in_ref, v_in_ref, k_out_ref, v_out_ref, mask_ref, semk, semv):
    i = pl.program_id(0)
    mask_ref[...] = jnp.ones_like(mask_ref)
    sl = pl.ds(i * _RB, _RB)
    ck = pltpu.make_async_copy(k_in_ref, k_out_ref.at[sl], semk)
    cv = pltpu.make_async_copy(v_in_ref, v_out_ref.at[sl], semv)
    ck.start()
    cv.start()
    ck.wait()
    cv.wait()


def kernel(k_val, v_val, input_pos, is_prefill, k_cache, v_cache, pos, mask):
    del input_pos, is_prefill, k_cache, v_cache, pos
    kv3 = (B * H, L, D)
    k3 = k_val.reshape(kv3)
    v3 = v_val.reshape(kv3)
    mask3 = (B * H, 1, L)
    k_out, v_out, mask_out = pl.pallas_call(
        _fill_kernel,
        grid=(_G,),
        in_specs=[
            pl.BlockSpec((_RB, L, D), lambda i: (i, 0, 0)),
            pl.BlockSpec((_RB, L, D), lambda i: (i, 0, 0)),
        ],
        out_specs=[
            pl.BlockSpec(memory_space=pl.ANY),
            pl.BlockSpec(memory_space=pl.ANY),
            pl.BlockSpec((B * H, 1, L), lambda i: (0, 0, 0)),
        ],
        out_shape=[
            jax.ShapeDtypeStruct(kv3, k_val.dtype),
            jax.ShapeDtypeStruct(kv3, v_val.dtype),
            jax.ShapeDtypeStruct(mask3, jnp.bool_),
        ],
        scratch_shapes=[pltpu.SemaphoreType.DMA, pltpu.SemaphoreType.DMA],
    )(k3, v3)
    return (
        k_out.reshape(B, H, L, D),
        v_out.reshape(B, H, L, D),
        mask_out.reshape(B, H, 1, L),
    )
